# baseline (speedup 1.0000x reference)
import functools

import jax
import jax.numpy as jnp
from jax import lax
from jax.experimental import pallas as pl
from jax.experimental.pallas import tpu as pltpu
from jax.experimental.pallas import tpu_sc as plsc

_N = 100000
_S = 10
_D = 128
_B = 4096
_NC = 2
_NS = 16
_NW = _NC * _NS
_PB = _B // _NW
_L = 16
_CD = 8
_CR = _CD * _S
_NBUF = 4
_GC = 128
_NB = _B * _S

_CH = (_PB * _S + _PB * _S * _S) // _CR
_CH_N2 = _PB * _S * _S // _CR
_PAD = 2 * _GC // _CR


def _expand_ids(src, src_off, dst, n_src, src_padded):
    lane = lax.iota(jnp.int32, _L)
    zeros = jnp.zeros((_L,), jnp.int32)

    def body(m, carry):
        if src_padded:
            g5 = (m * 6554) >> 15
            s_off = src_off + g5 * _GC + (m - 5 * g5) * _L
        else:
            s_off = src_off + m * _L
        v = src[pl.ds(s_off, _L)]
        for tt in range(_S):
            j = lane + tt * _L
            pos = (j * 6554) >> 16
            slot = j - pos * _S
            g = jnp.take_along_axis(v, pos, axis=0,
                                    mode="promise_in_bounds")
            d_off = m * 2 * _GC + (tt * _L if tt < 5
                                   else _GC + (tt - 5) * _L)
            dst[pl.ds(d_off, _L)] = g * _S + slot
        for h in range(3):
            dst[pl.ds(m * 2 * _GC + _CR + h * _L, _L)] = zeros
            dst[pl.ds(m * 2 * _GC + _GC + _CR + h * _L, _L)] = zeros
        return carry

    lax.fori_loop(0, n_src // _L, body, 0)


def _elem_gather(table_hbm, idx_v, dst, dst_off, nchunks, sem):
    descs = []
    for c in range(nchunks):
        if c >= 10:
            descs[c - 10].wait()
        descs.append(pltpu.async_copy(
            table_hbm.at[idx_v.at[pl.ds(c * _GC, _GC)]],
            dst.at[pl.ds(dst_off + c * _GC, _GC)], sem))
    for c in range(max(0, nchunks - 10), nchunks):
        descs[c].wait()


@functools.partial(
    pl.kernel,
    out_type=(
        jax.ShapeDtypeStruct((_B * (_S + 1), _D), jnp.float32),
        jax.ShapeDtypeStruct((_B * (_S + 1), _D), jnp.float32),
    ),
    mesh=plsc.VectorSubcoreMesh(core_axis_name="c", subcore_axis_name="s"),
    scratch_types=[
        pltpu.VMEM((_PB,), jnp.int32),
        pltpu.VMEM((_PB * _L,), jnp.int32),
        pltpu.VMEM((_PB * _S * _L,), jnp.int32),
        pltpu.VMEM((_PB * (_S + 1) * _L,), jnp.int32),
        pltpu.VMEM((_PB, _D), jnp.float32),
        pltpu.VMEM((_CR, _D), jnp.float32),
        pltpu.VMEM((_CR, _D), jnp.float32),
        pltpu.VMEM((_CR, _D), jnp.float32),
        pltpu.VMEM((_CR, _D), jnp.float32),
        pltpu.VMEM((_CD, _D), jnp.float32),
        pltpu.VMEM((_CD, _D), jnp.float32),
        pltpu.SemaphoreType.DMA,
        pltpu.SemaphoreType.DMA,
        pltpu.SemaphoreType.DMA,
        pltpu.SemaphoreType.DMA,
    ],
)
def _sc_gather(nodes_hbm, neigh_flat_hbm, feat_hbm,
               self_all, agg_all,
               nb_v, e1, e2, nall, gbuf, sbuf0, sbuf1, sbuf2, sbuf3,
               acc0, acc1,
               sem_idx, sem_feat, sem_self, sem_aggout):
    wid = lax.axis_index("s") * _NC + lax.axis_index("c")
    base = wid * _PB
    nbase = wid * _PB * _S
    n1_off = _PB * _S * _L
    sbufs = (sbuf0, sbuf1, sbuf2, sbuf3)
    accs = (acc0, acc1)

    pltpu.sync_copy(nodes_hbm.at[pl.ds(base, _PB)], nb_v)
    _expand_ids(nb_v, 0, e1, _PB, src_padded=False)
    _elem_gather(neigh_flat_hbm, e1, nall, n1_off,
                 _PB * _L // _GC, sem_idx)
    _expand_ids(nall, n1_off, e2, _PB * _S, src_padded=True)
    _elem_gather(neigh_flat_hbm, e2, nall, 0,
                 _PB * _S * _L // _GC, sem_idx)

    pltpu.async_copy(feat_hbm.at[nb_v], gbuf, sem_idx).wait()
    pltpu.async_copy(gbuf, self_all.at[pl.ds(_NB + base, _PB)],
                     sem_self).wait()

    def _gather_args(c, b):
        return (feat_hbm.at[nall.at[pl.ds(c * _GC, _CR)]], sbufs[b],
                sem_feat)

    for b in range(_NBUF):
        pltpu.async_copy(*_gather_args(b, b))

    def main_step(p, carry):
        c0 = p * _NBUF
        for b in range(_NBUF):
            c = c0 + b
            pltpu.make_async_copy(*_gather_args(c, b)).wait()
            is_n1 = c >= _CH_N2

            @pl.when(is_n1)
            def _():
                pltpu.async_copy(
                    sbufs[b],
                    self_all.at[pl.ds(nbase + (c - _CH_N2) * _CR, _CR)],
                    sem_self)

            @pl.when(c >= 2)
            def _():
                pltpu.make_async_copy(
                    accs[b % 2], agg_all.at[pl.ds(nbase, _CD)],
                    sem_aggout).wait()

            ab = sbufs[b]
            ac = accs[b % 2]
            for d in range(_CD):
                for v in range(_D // _L):
                    acc = ab[d * _S, pl.ds(v * _L, _L)]
                    for s in range(1, _S):
                        acc = acc + ab[d * _S + s, pl.ds(v * _L, _L)]
                    ac[d, pl.ds(v * _L, _L)] = acc
            out_row = jnp.where(c < _CH_N2,
                                nbase + c * _CD,
                                _NB + base + (c - _CH_N2) * _CD)
            pltpu.async_copy(ac, agg_all.at[pl.ds(out_row, _CD)],
                             sem_aggout)

            @pl.when(is_n1)
            def _():
                pltpu.make_async_copy(
                    sbufs[b], self_all.at[pl.ds(nbase, _CR)],
                    sem_self).wait()

            @pl.when(c + _NBUF < _CH)
            def _():
                pltpu.async_copy(*_gather_args(c + _NBUF, b))

        return carry

    lax.fori_loop(0, _CH // _NBUF, main_step, 0)

    for _ in range(2):
        pltpu.make_async_copy(acc0, agg_all.at[pl.ds(nbase, _CD)],
                              sem_aggout).wait()


_PT = 512


def _tc_block(self_b_ref, agg_b_ref, self_n_ref, agg_n_ref, w1_ref, w2_ref,
              out_ref):
    dn = (((1,), (1,)), ((), ()))
    w1s = w1_ref[:, :_D]
    w1n = w1_ref[:, _D:]
    w2s = w2_ref[:, :_D]
    w2n = w2_ref[:, _D:]
    inv_s = 1.0 / _S
    h1b = lax.dot_general(self_b_ref[...], w1s, dn,
                          preferred_element_type=jnp.float32)
    h1b += lax.dot_general(agg_b_ref[...] * inv_s, w1n, dn,
                           preferred_element_type=jnp.float32)
    h1b = jnp.maximum(h1b, 0.0)
    h1n = lax.dot_general(self_n_ref[...], w1s, dn,
                          preferred_element_type=jnp.float32)
    h1n += lax.dot_general(agg_n_ref[...] * inv_s, w1n, dn,
                           preferred_element_type=jnp.float32)
    h1n = jnp.maximum(h1n, 0.0)
    agg2 = jnp.sum(h1n.reshape(_PT, _S, _D), axis=1) * inv_s
    h2 = lax.dot_general(h1b, w2s, dn, preferred_element_type=jnp.float32)
    h2 += lax.dot_general(agg2, w2n, dn, preferred_element_type=jnp.float32)
    out_ref[...] = jnp.maximum(h2, 0.0)


def _tc_forward(self_all, agg_all, W1, W2):
    batch_spec = pl.BlockSpec((_PT, _D), lambda i: (i + _NB // _PT, 0))
    neigh_spec = pl.BlockSpec((_PT * _S, _D), lambda i: (i, 0))
    w_spec = pl.BlockSpec((_D, 2 * _D), lambda i: (0, 0))
    return pl.pallas_call(
        _tc_block,
        grid=(_B // _PT,),
        in_specs=[batch_spec, batch_spec, neigh_spec, neigh_spec,
                  w_spec, w_spec],
        out_specs=pl.BlockSpec((_PT, _D), lambda i: (i, 0)),
        out_shape=jax.ShapeDtypeStruct((_B, _D), jnp.float32),
    )(self_all, agg_all, self_all, agg_all, W1, W2)


def kernel(nodes_batch, neigh_idx, raw_features, W1, W2):
    neigh_flat = neigh_idx.reshape(-1)
    self_all, agg_all = _sc_gather(nodes_batch, neigh_flat, raw_features)
    return _tc_forward(self_all, agg_all, W1, W2)


# speedup vs baseline: 3.1728x; 3.1728x over previous
import functools

import jax
import jax.numpy as jnp
from jax import lax
from jax.experimental import pallas as pl
from jax.experimental.pallas import tpu as pltpu
from jax.experimental.pallas import tpu_sc as plsc

_N = 100000
_S = 10
_D = 128
_B = 4096
_NC = 2
_NS = 16
_NW = _NC * _NS
_PB = _B // _NW
_L = 16
_CD = 8
_CR = _CD * _S
_NBUF = 4
_GC = 128
_NB = _B * _S

_CH = (_PB * _S + _PB * _S * _S) // _CR
_CH_N2 = _PB * _S * _S // _CR
_PAD = 2 * _GC // _CR


def _expand_ids(src, src_off, dst, n_src, src_padded):
    lane = lax.iota(jnp.int32, _L)

    def body(m, carry):
        if src_padded:
            g5 = (m * 6554) >> 15
            s_off = src_off + g5 * _GC + (m - 5 * g5) * _L
        else:
            s_off = src_off + m * _L
        v = src[pl.ds(s_off, _L)]
        for tt in range(_S):
            j = lane + tt * _L
            pos = (j * 6554) >> 16
            slot = j - pos * _S
            g = jnp.take_along_axis(v, pos, axis=0,
                                    mode="promise_in_bounds")
            d_off = m * 2 * _GC + (tt * _L if tt < 5
                                   else _GC + (tt - 5) * _L)
            dst[pl.ds(d_off, _L)] = g * _S + slot
        return carry

    lax.fori_loop(0, n_src // _L, body, 0)


def _elem_gather(table_hbm, idx_v, dst, dst_off, nchunks, sem):
    descs = []
    for c in range(nchunks):
        if c >= 10:
            descs[c - 10].wait()
        descs.append(pltpu.async_copy(
            table_hbm.at[idx_v.at[pl.ds(c * _GC, _CR)]],
            dst.at[pl.ds(dst_off + c * _GC, _CR)], sem))
    for c in range(max(0, nchunks - 10), nchunks):
        descs[c].wait()


@functools.partial(
    pl.kernel,
    out_type=(
        jax.ShapeDtypeStruct((_B * (_S + 1), _D), jnp.float32),
        jax.ShapeDtypeStruct((_B * (_S + 1), _D), jnp.float32),
    ),
    mesh=plsc.VectorSubcoreMesh(core_axis_name="c", subcore_axis_name="s"),
    scratch_types=[
        pltpu.VMEM((_PB,), jnp.int32),
        pltpu.VMEM((_PB * _L,), jnp.int32),
        pltpu.VMEM((_PB * _S * _L,), jnp.int32),
        pltpu.VMEM((_PB * (_S + 1) * _L,), jnp.int32),
        pltpu.VMEM((_PB, _D), jnp.float32),
        pltpu.VMEM((_CR, _D), jnp.float32),
        pltpu.VMEM((_CR, _D), jnp.float32),
        pltpu.VMEM((_CR, _D), jnp.float32),
        pltpu.VMEM((_CR, _D), jnp.float32),
        pltpu.VMEM((_CD, _D), jnp.float32),
        pltpu.VMEM((_CD, _D), jnp.float32),
        pltpu.SemaphoreType.DMA,
        pltpu.SemaphoreType.DMA,
        pltpu.SemaphoreType.DMA,
        pltpu.SemaphoreType.DMA,
    ],
)
def _sc_gather(nodes_hbm, neigh_flat_hbm, feat_hbm,
               self_all, agg_all,
               nb_v, e1, e2, nall, gbuf, sbuf0, sbuf1, sbuf2, sbuf3,
               acc0, acc1,
               sem_idx, sem_feat, sem_self, sem_aggout):
    wid = lax.axis_index("s") * _NC + lax.axis_index("c")
    base = wid * _PB
    nbase = wid * _PB * _S
    n1_off = _PB * _S * _L
    sbufs = (sbuf0, sbuf1, sbuf2, sbuf3)
    accs = (acc0, acc1)

    pltpu.sync_copy(nodes_hbm.at[pl.ds(base, _PB)], nb_v)
    _expand_ids(nb_v, 0, e1, _PB, src_padded=False)
    _elem_gather(neigh_flat_hbm, e1, nall, n1_off,
                 _PB * _S // _CR, sem_idx)
    _expand_ids(nall, n1_off, e2, _PB * _S, src_padded=True)
    _elem_gather(neigh_flat_hbm, e2, nall, 0,
                 _PB * _S * _S // _CR, sem_idx)

    pltpu.async_copy(feat_hbm.at[nb_v], gbuf, sem_idx).wait()
    pltpu.async_copy(gbuf, self_all.at[pl.ds(_NB + base, _PB)],
                     sem_self).wait()

    def _gather_args(c, b):
        return (feat_hbm.at[nall.at[pl.ds(c * _GC, _CR)]], sbufs[b],
                sem_feat)

    for b in range(_NBUF):
        pltpu.async_copy(*_gather_args(b, b))

    def main_step(p, carry):
        c0 = p * _NBUF
        for b in range(_NBUF):
            c = c0 + b
            pltpu.make_async_copy(*_gather_args(c, b)).wait()
            is_n1 = c >= _CH_N2

            @pl.when(is_n1)
            def _():
                pltpu.async_copy(
                    sbufs[b],
                    self_all.at[pl.ds(nbase + (c - _CH_N2) * _CR, _CR)],
                    sem_self)

            @pl.when(c >= 2)
            def _():
                pltpu.make_async_copy(
                    accs[b % 2], agg_all.at[pl.ds(nbase, _CD)],
                    sem_aggout).wait()

            ab = sbufs[b]
            ac = accs[b % 2]
            for d in range(_CD):
                for v in range(_D // _L):
                    acc = ab[d * _S, pl.ds(v * _L, _L)]
                    for s in range(1, _S):
                        acc = acc + ab[d * _S + s, pl.ds(v * _L, _L)]
                    ac[d, pl.ds(v * _L, _L)] = acc
            out_row = jnp.where(c < _CH_N2,
                                nbase + c * _CD,
                                _NB + base + (c - _CH_N2) * _CD)
            pltpu.async_copy(ac, agg_all.at[pl.ds(out_row, _CD)],
                             sem_aggout)

            @pl.when(is_n1)
            def _():
                pltpu.make_async_copy(
                    sbufs[b], self_all.at[pl.ds(nbase, _CR)],
                    sem_self).wait()

            @pl.when(c + _NBUF < _CH)
            def _():
                pltpu.async_copy(*_gather_args(c + _NBUF, b))

        return carry

    lax.fori_loop(0, _CH // _NBUF, main_step, 0)

    for _ in range(2):
        pltpu.make_async_copy(acc0, agg_all.at[pl.ds(nbase, _CD)],
                              sem_aggout).wait()


_PT = 512


def _tc_block(self_b_ref, agg_b_ref, self_n_ref, agg_n_ref, w1_ref, w2_ref,
              out_ref):
    dn = (((1,), (1,)), ((), ()))
    w1s = w1_ref[:, :_D]
    w1n = w1_ref[:, _D:]
    w2s = w2_ref[:, :_D]
    w2n = w2_ref[:, _D:]
    inv_s = 1.0 / _S
    h1b = lax.dot_general(self_b_ref[...], w1s, dn,
                          preferred_element_type=jnp.float32)
    h1b += lax.dot_general(agg_b_ref[...] * inv_s, w1n, dn,
                           preferred_element_type=jnp.float32)
    h1b = jnp.maximum(h1b, 0.0)
    h1n = lax.dot_general(self_n_ref[...], w1s, dn,
                          preferred_element_type=jnp.float32)
    h1n += lax.dot_general(agg_n_ref[...] * inv_s, w1n, dn,
                           preferred_element_type=jnp.float32)
    h1n = jnp.maximum(h1n, 0.0)
    agg2 = jnp.sum(h1n.reshape(_PT, _S, _D), axis=1) * inv_s
    h2 = lax.dot_general(h1b, w2s, dn, preferred_element_type=jnp.float32)
    h2 += lax.dot_general(agg2, w2n, dn, preferred_element_type=jnp.float32)
    out_ref[...] = jnp.maximum(h2, 0.0)


def _tc_forward(self_all, agg_all, W1, W2):
    batch_spec = pl.BlockSpec((_PT, _D), lambda i: (i + _NB // _PT, 0))
    neigh_spec = pl.BlockSpec((_PT * _S, _D), lambda i: (i, 0))
    w_spec = pl.BlockSpec((_D, 2 * _D), lambda i: (0, 0))
    return pl.pallas_call(
        _tc_block,
        grid=(_B // _PT,),
        in_specs=[batch_spec, batch_spec, neigh_spec, neigh_spec,
                  w_spec, w_spec],
        out_specs=pl.BlockSpec((_PT, _D), lambda i: (i, 0)),
        out_shape=jax.ShapeDtypeStruct((_B, _D), jnp.float32),
    )(self_all, agg_all, self_all, agg_all, W1, W2)


def kernel(nodes_batch, neigh_idx, raw_features, W1, W2):
    neigh_flat = neigh_idx.reshape(-1)
    self_all, agg_all = _sc_gather(nodes_batch, neigh_flat, raw_features)
    return _tc_forward(self_all, agg_all, W1, W2)


# speedup vs baseline: 3.1966x; 1.0075x over previous
import functools

import jax
import jax.numpy as jnp
from jax import lax
from jax.experimental import pallas as pl
from jax.experimental.pallas import tpu as pltpu
from jax.experimental.pallas import tpu_sc as plsc

_N = 100000
_S = 10
_D = 128
_B = 4096
_NC = 2
_NS = 16
_NW = _NC * _NS
_PB = _B // _NW
_L = 16
_CD = 8
_CR = _CD * _S
_NBUF = 4
_GC = 128
_NB = _B * _S

_N1_CH = _PB * _S // _GC
_N2_CH = _PB * _S * _S // _GC
_CH = (_PB * _S + _PB * _S * _S) // _CR
_CH_N2 = _PB * _S * _S // _CR


def _expand_ids(src, src_off, dst, n_src):
    lane = lax.iota(jnp.int32, _L)

    def body(m, carry):
        v = src[pl.ds(src_off + m * _L, _L)]
        for tt in range(_S):
            j = lane + tt * _L
            pos = (j * 6554) >> 16
            slot = j - pos * _S
            g = jnp.take_along_axis(v, pos, axis=0,
                                    mode="promise_in_bounds")
            dst[pl.ds(m * _L * _S + tt * _L, _L)] = g * _S + slot
        return carry

    lax.fori_loop(0, n_src // _L, body, 0)


def _elem_gather(table_hbm, idx_v, dst, dst_off, nchunks, sem):
    descs = []
    for c in range(nchunks):
        if c >= 10:
            descs[c - 10].wait()
        descs.append(pltpu.async_copy(
            table_hbm.at[idx_v.at[pl.ds(c * _GC, _GC)]],
            dst.at[pl.ds(dst_off + c * _GC, _GC)], sem))
    for c in range(max(0, nchunks - 10), nchunks):
        descs[c].wait()


@functools.partial(
    pl.kernel,
    out_type=(
        jax.ShapeDtypeStruct((_B * (_S + 1), _D), jnp.float32),
        jax.ShapeDtypeStruct((_B * (_S + 1), _D), jnp.float32),
    ),
    mesh=plsc.VectorSubcoreMesh(core_axis_name="c", subcore_axis_name="s"),
    scratch_types=[
        pltpu.VMEM((_PB,), jnp.int32),
        pltpu.VMEM((_PB * _S,), jnp.int32),
        pltpu.VMEM((_PB * _S * _S,), jnp.int32),
        pltpu.VMEM((_PB * _S * (_S + 1),), jnp.int32),
        pltpu.VMEM((_PB, _D), jnp.float32),
        pltpu.VMEM((_CR, _D), jnp.float32),
        pltpu.VMEM((_CR, _D), jnp.float32),
        pltpu.VMEM((_CR, _D), jnp.float32),
        pltpu.VMEM((_CR, _D), jnp.float32),
        pltpu.VMEM((_CD, _D), jnp.float32),
        pltpu.VMEM((_CD, _D), jnp.float32),
        pltpu.SemaphoreType.DMA,
        pltpu.SemaphoreType.DMA,
        pltpu.SemaphoreType.DMA,
        pltpu.SemaphoreType.DMA,
    ],
)
def _sc_gather(nodes_hbm, neigh_flat_hbm, feat_hbm,
               self_all, agg_all,
               nb_v, e1, e2, nall, gbuf, sbuf0, sbuf1, sbuf2, sbuf3,
               acc0, acc1,
               sem_idx, sem_feat, sem_self, sem_aggout):
    wid = lax.axis_index("s") * _NC + lax.axis_index("c")
    base = wid * _PB
    nbase = wid * _PB * _S
    n1_off = _PB * _S * _S
    sbufs = (sbuf0, sbuf1, sbuf2, sbuf3)
    accs = (acc0, acc1)

    pltpu.sync_copy(nodes_hbm.at[pl.ds(base, _PB)], nb_v)
    _expand_ids(nb_v, 0, e1, _PB)
    _elem_gather(neigh_flat_hbm, e1, nall, n1_off, _N1_CH, sem_idx)
    _expand_ids(nall, n1_off, e2, _PB * _S)
    _elem_gather(neigh_flat_hbm, e2, nall, 0, _N2_CH, sem_idx)

    pltpu.async_copy(feat_hbm.at[nb_v], gbuf, sem_idx).wait()
    pltpu.async_copy(gbuf, self_all.at[pl.ds(_NB + base, _PB)],
                     sem_self).wait()

    def _gather_args(c, b):
        return (feat_hbm.at[nall.at[pl.ds(c * _CR, _CR)]], sbufs[b],
                sem_feat)

    for b in range(_NBUF):
        pltpu.async_copy(*_gather_args(b, b))

    def main_step(p, carry):
        c0 = p * _NBUF
        for b in range(_NBUF):
            c = c0 + b
            pltpu.make_async_copy(*_gather_args(c, b)).wait()
            is_n1 = c >= _CH_N2

            @pl.when(is_n1)
            def _():
                pltpu.async_copy(
                    sbufs[b],
                    self_all.at[pl.ds(nbase + (c - _CH_N2) * _CR, _CR)],
                    sem_self)

            @pl.when(c >= 2)
            def _():
                pltpu.make_async_copy(
                    accs[b % 2], agg_all.at[pl.ds(nbase, _CD)],
                    sem_aggout).wait()

            ab = sbufs[b]
            ac = accs[b % 2]
            for d in range(_CD):
                for v in range(_D // _L):
                    acc = ab[d * _S, pl.ds(v * _L, _L)]
                    for s in range(1, _S):
                        acc = acc + ab[d * _S + s, pl.ds(v * _L, _L)]
                    ac[d, pl.ds(v * _L, _L)] = acc
            out_row = jnp.where(c < _CH_N2,
                                nbase + c * _CD,
                                _NB + base + (c - _CH_N2) * _CD)
            pltpu.async_copy(ac, agg_all.at[pl.ds(out_row, _CD)],
                             sem_aggout)

            @pl.when(is_n1)
            def _():
                pltpu.make_async_copy(
                    sbufs[b], self_all.at[pl.ds(nbase, _CR)],
                    sem_self).wait()

            @pl.when(c + _NBUF < _CH)
            def _():
                pltpu.async_copy(*_gather_args(c + _NBUF, b))

        return carry

    lax.fori_loop(0, _CH // _NBUF, main_step, 0)

    for _ in range(2):
        pltpu.make_async_copy(acc0, agg_all.at[pl.ds(nbase, _CD)],
                              sem_aggout).wait()


_PT = 512


def _tc_block(self_b_ref, agg_b_ref, self_n_ref, agg_n_ref, w1_ref, w2_ref,
              out_ref):
    dn = (((1,), (1,)), ((), ()))
    w1s = w1_ref[:, :_D]
    w1n = w1_ref[:, _D:]
    w2s = w2_ref[:, :_D]
    w2n = w2_ref[:, _D:]
    inv_s = 1.0 / _S
    h1b = lax.dot_general(self_b_ref[...], w1s, dn,
                          preferred_element_type=jnp.float32)
    h1b += lax.dot_general(agg_b_ref[...] * inv_s, w1n, dn,
                           preferred_element_type=jnp.float32)
    h1b = jnp.maximum(h1b, 0.0)
    h1n = lax.dot_general(self_n_ref[...], w1s, dn,
                          preferred_element_type=jnp.float32)
    h1n += lax.dot_general(agg_n_ref[...] * inv_s, w1n, dn,
                           preferred_element_type=jnp.float32)
    h1n = jnp.maximum(h1n, 0.0)
    agg2 = jnp.sum(h1n.reshape(_PT, _S, _D), axis=1) * inv_s
    h2 = lax.dot_general(h1b, w2s, dn, preferred_element_type=jnp.float32)
    h2 += lax.dot_general(agg2, w2n, dn, preferred_element_type=jnp.float32)
    out_ref[...] = jnp.maximum(h2, 0.0)


def _tc_forward(self_all, agg_all, W1, W2):
    batch_spec = pl.BlockSpec((_PT, _D), lambda i: (i + _NB // _PT, 0))
    neigh_spec = pl.BlockSpec((_PT * _S, _D), lambda i: (i, 0))
    w_spec = pl.BlockSpec((_D, 2 * _D), lambda i: (0, 0))
    return pl.pallas_call(
        _tc_block,
        grid=(_B // _PT,),
        in_specs=[batch_spec, batch_spec, neigh_spec, neigh_spec,
                  w_spec, w_spec],
        out_specs=pl.BlockSpec((_PT, _D), lambda i: (i, 0)),
        out_shape=jax.ShapeDtypeStruct((_B, _D), jnp.float32),
    )(self_all, agg_all, self_all, agg_all, W1, W2)


def kernel(nodes_batch, neigh_idx, raw_features, W1, W2):
    neigh_flat = neigh_idx.reshape(-1)
    self_all, agg_all = _sc_gather(nodes_batch, neigh_flat, raw_features)
    return _tc_forward(self_all, agg_all, W1, W2)


# speedup vs baseline: 3.3777x; 1.0567x over previous
import functools

import jax
import jax.numpy as jnp
from jax import lax
from jax.experimental import pallas as pl
from jax.experimental.pallas import tpu as pltpu
from jax.experimental.pallas import tpu_sc as plsc

_N = 100000
_S = 10
_D = 128
_B = 4096
_NC = 2
_NS = 16
_NW = _NC * _NS
_PB = _B // _NW
_L = 16
_CD = 8
_CR = _CD * _S
_NBUF = 4
_GC = 128
_NB = _B * _S

_N1_CH = _PB * _S // _GC
_N2_CH = _PB * _S * _S // _GC
_CH = (_PB * _S + _PB * _S * _S) // _CR
_CH_N2 = _PB * _S * _S // _CR


def _expand_ids(src, src_off, dst, n_src):
    lane = lax.iota(jnp.int32, _L)

    def body(m, carry):
        v = src[pl.ds(src_off + m * _L, _L)]
        for tt in range(_S):
            j = lane + tt * _L
            pos = (j * 6554) >> 16
            slot = j - pos * _S
            g = jnp.take_along_axis(v, pos, axis=0,
                                    mode="promise_in_bounds")
            dst[pl.ds(m * _L * _S + tt * _L, _L)] = g * _S + slot
        return carry

    lax.fori_loop(0, n_src // _L, body, 0)


def _elem_gather(table_hbm, idx_v, dst, dst_off, nchunks, sem):
    descs = []
    for c in range(nchunks):
        if c >= 10:
            descs[c - 10].wait()
        descs.append(pltpu.async_copy(
            table_hbm.at[idx_v.at[pl.ds(c * _GC, _GC)]],
            dst.at[pl.ds(dst_off + c * _GC, _GC)], sem))
    for c in range(max(0, nchunks - 10), nchunks):
        descs[c].wait()


@functools.partial(
    pl.kernel,
    out_type=(
        jax.ShapeDtypeStruct((_B * (_S + 1), _D), jnp.float32),
        jax.ShapeDtypeStruct((_B * (_S + 1), _D), jnp.float32),
    ),
    mesh=plsc.VectorSubcoreMesh(core_axis_name="c", subcore_axis_name="s"),
    scratch_types=[
        pltpu.VMEM((_PB,), jnp.int32),
        pltpu.VMEM((_PB * _S,), jnp.int32),
        pltpu.VMEM((_PB * _S * _S,), jnp.int32),
        pltpu.VMEM((_PB * _S * (_S + 1),), jnp.int32),
        pltpu.VMEM((_PB, _D), jnp.float32),
        pltpu.VMEM((_CR, _D), jnp.float32),
        pltpu.VMEM((_CR, _D), jnp.float32),
        pltpu.VMEM((_CR, _D), jnp.float32),
        pltpu.VMEM((_CR, _D), jnp.float32),
        pltpu.VMEM((_CD, _D), jnp.float32),
        pltpu.VMEM((_CD, _D), jnp.float32),
        pltpu.SemaphoreType.DMA,
        pltpu.SemaphoreType.DMA,
        pltpu.SemaphoreType.DMA,
        pltpu.SemaphoreType.DMA,
    ],
)
def _sc_gather(nodes_hbm, neigh_flat_hbm, feat_hbm,
               self_all, agg_all,
               nb_v, e1, e2, nall, gbuf, sbuf0, sbuf1, sbuf2, sbuf3,
               acc0, acc1,
               sem_idx, sem_feat, sem_self, sem_aggout):
    wid = lax.axis_index("s") * _NC + lax.axis_index("c")
    base = wid * _PB
    nbase = wid * _PB * _S
    n1_off = _PB * _S * _S
    sbufs = (sbuf0, sbuf1, sbuf2, sbuf3)
    accs = (acc0, acc1)

    pltpu.sync_copy(nodes_hbm.at[pl.ds(base, _PB)], nb_v)
    _expand_ids(nb_v, 0, e1, _PB)
    _elem_gather(neigh_flat_hbm, e1, nall, n1_off, _N1_CH, sem_idx)
    _expand_ids(nall, n1_off, e2, _PB * _S)
    _elem_gather(neigh_flat_hbm, e2, nall, 0, _N2_CH, sem_idx)

    pltpu.async_copy(feat_hbm.at[nb_v], gbuf, sem_idx).wait()
    pltpu.async_copy(gbuf, self_all.at[pl.ds(_NB + base, _PB)],
                     sem_self).wait()

    def _gather_args(c, b):
        return (feat_hbm.at[nall.at[pl.ds(c * _CR, _CR)]], sbufs[b],
                sem_feat)

    for b in range(_NBUF):
        pltpu.async_copy(*_gather_args(b, b))

    def main_step(p, carry):
        c0 = p * _NBUF
        for b in range(_NBUF):
            c = c0 + b
            pltpu.make_async_copy(*_gather_args(c, b)).wait()
            is_n1 = c >= _CH_N2

            @pl.when(is_n1)
            def _():
                pltpu.async_copy(
                    sbufs[b],
                    self_all.at[pl.ds(nbase + (c - _CH_N2) * _CR, _CR)],
                    sem_self)

            @pl.when(c >= 2)
            def _():
                pltpu.make_async_copy(
                    accs[b % 2], agg_all.at[pl.ds(nbase, _CD)],
                    sem_aggout).wait()

            ab = sbufs[b]
            ac = accs[b % 2]
            for d in range(_CD):
                for v in range(_D // _L):
                    sl = pl.ds(v * _L, _L)
                    x = [ab[d * _S + s, sl] for s in range(_S)]
                    t0 = x[0] + x[1]
                    t1 = x[2] + x[3]
                    t2 = x[4] + x[5]
                    t3 = x[6] + x[7]
                    t4 = x[8] + x[9]
                    ac[d, sl] = ((t0 + t1) + (t2 + t3)) + t4
            out_row = jnp.where(c < _CH_N2,
                                nbase + c * _CD,
                                _NB + base + (c - _CH_N2) * _CD)
            pltpu.async_copy(ac, agg_all.at[pl.ds(out_row, _CD)],
                             sem_aggout)

            @pl.when(is_n1)
            def _():
                pltpu.make_async_copy(
                    sbufs[b], self_all.at[pl.ds(nbase, _CR)],
                    sem_self).wait()

            @pl.when(c + _NBUF < _CH)
            def _():
                pltpu.async_copy(*_gather_args(c + _NBUF, b))

        return carry

    lax.fori_loop(0, _CH // _NBUF, main_step, 0)

    for _ in range(2):
        pltpu.make_async_copy(acc0, agg_all.at[pl.ds(nbase, _CD)],
                              sem_aggout).wait()


_PT = 512


def _tc_block(self_b_ref, agg_b_ref, self_n_ref, agg_n_ref, w1_ref, w2_ref,
              out_ref):
    dn = (((1,), (1,)), ((), ()))
    w1s = w1_ref[:, :_D]
    w1n = w1_ref[:, _D:]
    w2s = w2_ref[:, :_D]
    w2n = w2_ref[:, _D:]
    inv_s = 1.0 / _S
    h1b = lax.dot_general(self_b_ref[...], w1s, dn,
                          preferred_element_type=jnp.float32)
    h1b += lax.dot_general(agg_b_ref[...] * inv_s, w1n, dn,
                           preferred_element_type=jnp.float32)
    h1b = jnp.maximum(h1b, 0.0)
    h1n = lax.dot_general(self_n_ref[...], w1s, dn,
                          preferred_element_type=jnp.float32)
    h1n += lax.dot_general(agg_n_ref[...] * inv_s, w1n, dn,
                           preferred_element_type=jnp.float32)
    h1n = jnp.maximum(h1n, 0.0)
    agg2 = jnp.sum(h1n.reshape(_PT, _S, _D), axis=1) * inv_s
    h2 = lax.dot_general(h1b, w2s, dn, preferred_element_type=jnp.float32)
    h2 += lax.dot_general(agg2, w2n, dn, preferred_element_type=jnp.float32)
    out_ref[...] = jnp.maximum(h2, 0.0)


def _tc_forward(self_all, agg_all, W1, W2):
    batch_spec = pl.BlockSpec((_PT, _D), lambda i: (i + _NB // _PT, 0))
    neigh_spec = pl.BlockSpec((_PT * _S, _D), lambda i: (i, 0))
    w_spec = pl.BlockSpec((_D, 2 * _D), lambda i: (0, 0))
    return pl.pallas_call(
        _tc_block,
        grid=(_B // _PT,),
        in_specs=[batch_spec, batch_spec, neigh_spec, neigh_spec,
                  w_spec, w_spec],
        out_specs=pl.BlockSpec((_PT, _D), lambda i: (i, 0)),
        out_shape=jax.ShapeDtypeStruct((_B, _D), jnp.float32),
    )(self_all, agg_all, self_all, agg_all, W1, W2)


def kernel(nodes_batch, neigh_idx, raw_features, W1, W2):
    neigh_flat = neigh_idx.reshape(-1)
    self_all, agg_all = _sc_gather(nodes_batch, neigh_flat, raw_features)
    return _tc_forward(self_all, agg_all, W1, W2)
